# baseline (speedup 1.0000x reference)
import functools

import jax
import jax.numpy as jnp
from jax import lax
from jax.experimental import pallas as pl
from jax.experimental.pallas import tpu as pltpu
from jax.experimental.pallas import tpu_sc as plsc

N = 10000
D = 128
E = 320000
NC = 2
NS = 16
NW = NC * NS
CHUNK = 32
CHUNKS = 320
BC = 80
NBUF = 8
EPW = CHUNKS * CHUNK
E_PAD = NW * EPW
N_PAD = 10240
RPT = N_PAD // NS

_mesh = plsc.VectorSubcoreMesh(
    core_axis_name="c", subcore_axis_name="s", num_cores=NC, num_subcores=NS
)


@functools.partial(
    pl.kernel,
    out_type=jax.ShapeDtypeStruct((NC * N_PAD,), jnp.float32),
    mesh=_mesh,
    scratch_types=[
        pltpu.VMEM((CHUNKS, CHUNK), jnp.int32),
        pltpu.VMEM((CHUNK,), jnp.float32),
        pltpu.VMEM((RPT,), jnp.float32),
        pltpu.VMEM_SHARED((N_PAD,), jnp.float32),
    ],
)
def _sc_degree(dst_hbm, zeros1_hbm, ones_hbm, deg_out, dstm, onesv, stage, deg_sh):
    c = lax.axis_index("c")
    s = lax.axis_index("s")
    wid = c * NS + s
    pltpu.sync_copy(zeros1_hbm.at[pl.ds(s * RPT, RPT)], stage)
    pltpu.sync_copy(stage, deg_sh.at[pl.ds(s * RPT, RPT)])
    pltpu.sync_copy(ones_hbm, onesv)
    pltpu.sync_copy(dst_hbm.at[wid], dstm)
    plsc.subcore_barrier()

    def body(i, carry):
        pltpu.sync_copy(onesv, deg_sh.at[dstm.at[i]], add=True)
        return carry

    lax.fori_loop(0, CHUNKS, body, 0, unroll=False)
    plsc.subcore_barrier()
    pltpu.sync_copy(deg_sh.at[pl.ds(s * RPT, RPT)], stage)
    pltpu.sync_copy(stage, deg_out.at[pl.ds(pl.multiple_of(c * N_PAD + s * RPT, 8), RPT)])


@functools.partial(
    pl.kernel,
    out_type=jax.ShapeDtypeStruct((NC, N_PAD, D), jnp.float32),
    mesh=_mesh,
    scratch_types=[
        pltpu.VMEM((BC * CHUNK,), jnp.int32),
        pltpu.VMEM((BC, CHUNK), jnp.int32),
        [pltpu.VMEM((CHUNK, D), jnp.float32) for _ in range(NBUF)],
        pltpu.VMEM_SHARED((N_PAD, D), jnp.float32),
        [pltpu.SemaphoreType.DMA for _ in range(NBUF)],
    ],
)
def _sc_agg(p_hbm, src_hbm, dst_hbm, zeros2_hbm, acc_out, srcv, dstm, rows,
            acc_sh, gsem):
    c = lax.axis_index("c")
    s = lax.axis_index("s")
    wid = c * NS + s
    pltpu.sync_copy(zeros2_hbm.at[pl.ds(s * RPT, RPT)],
                    acc_sh.at[pl.ds(s * RPT, RPT)])

    pltpu.sync_copy(src_hbm.at[wid, pl.ds(0, BC * CHUNK)], srcv)
    pltpu.sync_copy(dst_hbm.at[wid, pl.ds(0, BC)], dstm)

    def _gather(j, b):
        pltpu.async_copy(
            p_hbm.at[srcv.at[pl.ds(pl.multiple_of(j * CHUNK, 8), CHUNK)]],
            rows[b], gsem[b])

    def _wait(j, b):
        pltpu.make_async_copy(
            p_hbm.at[srcv.at[pl.ds(pl.multiple_of(j * CHUNK, 8), CHUNK)]],
            rows[b], gsem[b]).wait()

    for blk in range(CHUNKS // BC):
        if blk > 0:
            pltpu.sync_copy(
                src_hbm.at[wid, pl.ds(blk * BC * CHUNK, BC * CHUNK)], srcv)
            pltpu.sync_copy(dst_hbm.at[wid, pl.ds(blk * BC, BC)], dstm)
        for b in range(NBUF):
            _gather(b, b)
        if blk == 0:
            plsc.subcore_barrier()

        def group(g, carry):
            for b in range(NBUF):
                j = g * NBUF + b
                _wait(j, b)
                pltpu.sync_copy(rows[b], acc_sh.at[dstm.at[j]], add=True)
                nxt = j + NBUF

                @pl.when(nxt < BC)
                def _():
                    _gather(nxt, b)
            return carry

        lax.fori_loop(0, BC // NBUF, group, 0, unroll=False)
    plsc.subcore_barrier()
    pltpu.sync_copy(acc_sh.at[pl.ds(s * RPT, RPT)],
                    acc_out.at[c, pl.ds(s * RPT, RPT)])


BLK = 1024
GRID = N_PAD // BLK


def _dinv_of(deg_ref):
    deg = deg_ref[0] + deg_ref[1] + 1.0
    return lax.rsqrt(deg)


def _tc_mm_body(x_ref, w_ref, h_ref):
    h_ref[...] = jnp.dot(x_ref[...], w_ref[...],
                         preferred_element_type=jnp.float32)


def _tc_scale_body(deg_ref, h_ref, p_ref):
    p_ref[...] = h_ref[...] * _dinv_of(deg_ref)


def _tc_mid_body(deg_ref, acc_ref, p_ref, b_ref, w_ref, p2_ref):
    dinv = _dinv_of(deg_ref)
    pre = (acc_ref[0] + acc_ref[1] + p_ref[...]) * dinv + b_ref[...]
    mid = jnp.where(pre >= 0.0, pre, 0.01 * pre)
    h = jnp.dot(mid, w_ref[...], preferred_element_type=jnp.float32)
    p2_ref[...] = h * dinv


def _tc_out_body(deg_ref, acc_ref, p_ref, b_ref, o_ref):
    dinv = _dinv_of(deg_ref)
    o_ref[...] = (acc_ref[0] + acc_ref[1] + p_ref[...]) * dinv + b_ref[...]


_deg_spec = pl.BlockSpec((NC, BLK, 1), lambda i: (0, i, 0))
_acc_spec = pl.BlockSpec((NC, BLK, D), lambda i: (0, i, 0))
_row_spec = pl.BlockSpec((BLK, D), lambda i: (i, 0))
_w_spec = pl.BlockSpec((D, D), lambda i: (0, 0))
_b_spec = pl.BlockSpec((1, D), lambda i: (0, 0))
_pD = jax.ShapeDtypeStruct((N_PAD, D), jnp.float32)

_tc_mm = pl.pallas_call(
    _tc_mm_body, grid=(GRID,),
    in_specs=[_row_spec, _w_spec],
    out_specs=_row_spec, out_shape=_pD)

_tc_scale = pl.pallas_call(
    _tc_scale_body, grid=(GRID,),
    in_specs=[_deg_spec, _row_spec],
    out_specs=_row_spec, out_shape=_pD)

_tc_mid = pl.pallas_call(
    _tc_mid_body, grid=(GRID,),
    in_specs=[_deg_spec, _acc_spec, _row_spec, _b_spec, _w_spec],
    out_specs=_row_spec, out_shape=_pD)

_tc_out = pl.pallas_call(
    _tc_out_body, grid=(GRID,),
    in_specs=[_deg_spec, _acc_spec, _row_spec, _b_spec],
    out_specs=_row_spec, out_shape=_pD)


def kernel(x, edge_index, W1, b1, W2, b2):
    src = edge_index[0].astype(jnp.int32)
    dst = edge_index[1].astype(jnp.int32)
    pad = N + jnp.arange(E_PAD - E, dtype=jnp.int32) % (N_PAD - N)
    src2 = jnp.concatenate([src, pad]).reshape(NW, EPW)
    dst3 = jnp.concatenate([dst, pad]).reshape(NW, CHUNKS, CHUNK)
    x_pad = jnp.pad(x.astype(jnp.float32), ((0, N_PAD - N), (0, 0)))
    zeros1 = jnp.zeros((N_PAD,), jnp.float32)
    zeros2 = jnp.zeros((N_PAD, D), jnp.float32)
    ones_c = jnp.ones((CHUNK,), jnp.float32)

    h1 = _tc_mm(x_pad, W1.astype(jnp.float32))
    deg = _sc_degree(dst3, zeros1, ones_c).reshape(NC, N_PAD, 1)
    p1 = _tc_scale(deg, h1)
    acc1 = _sc_agg(p1, src2, dst3, zeros2)
    p2 = _tc_mid(deg, acc1, p1, b1.reshape(1, D).astype(jnp.float32),
                 W2.astype(jnp.float32))
    acc2 = _sc_agg(p2, src2, dst3, zeros2)
    out = _tc_out(deg, acc2, p2, b2.reshape(1, D).astype(jnp.float32))
    return out[:N]


# speedup vs baseline: 1.0976x; 1.0976x over previous
import functools

import jax
import jax.numpy as jnp
from jax import lax
from jax.experimental import pallas as pl
from jax.experimental.pallas import tpu as pltpu
from jax.experimental.pallas import tpu_sc as plsc

N = 10000
D = 128
E = 320000
NC = 2
NS = 16
NW = NC * NS
CHUNK = 64
CHUNKS = 160
BC = 80
NBUF = 4
EPW = CHUNKS * CHUNK
E_PAD = NW * EPW
N_PAD = 10240
RPT = N_PAD // NS

_mesh = plsc.VectorSubcoreMesh(
    core_axis_name="c", subcore_axis_name="s", num_cores=NC, num_subcores=NS
)


@functools.partial(
    pl.kernel,
    out_type=jax.ShapeDtypeStruct((NC * N_PAD,), jnp.float32),
    mesh=_mesh,
    scratch_types=[
        pltpu.VMEM((CHUNKS, CHUNK), jnp.int32),
        pltpu.VMEM((CHUNK,), jnp.float32),
        pltpu.VMEM((RPT,), jnp.float32),
        pltpu.VMEM_SHARED((N_PAD,), jnp.float32),
    ],
)
def _sc_degree(dst_hbm, zeros1_hbm, ones_hbm, deg_out, dstm, onesv, stage, deg_sh):
    c = lax.axis_index("c")
    s = lax.axis_index("s")
    wid = c * NS + s
    pltpu.sync_copy(zeros1_hbm.at[pl.ds(s * RPT, RPT)], stage)
    pltpu.sync_copy(stage, deg_sh.at[pl.ds(s * RPT, RPT)])
    pltpu.sync_copy(ones_hbm, onesv)
    pltpu.sync_copy(dst_hbm.at[wid], dstm)
    plsc.subcore_barrier()

    def body(i, carry):
        pltpu.sync_copy(onesv, deg_sh.at[dstm.at[i]], add=True)
        return carry

    lax.fori_loop(0, CHUNKS, body, 0, unroll=False)
    plsc.subcore_barrier()
    pltpu.sync_copy(deg_sh.at[pl.ds(s * RPT, RPT)], stage)
    pltpu.sync_copy(stage, deg_out.at[pl.ds(pl.multiple_of(c * N_PAD + s * RPT, 8), RPT)])


@functools.partial(
    pl.kernel,
    out_type=jax.ShapeDtypeStruct((NC, N_PAD, D), jnp.float32),
    mesh=_mesh,
    scratch_types=[
        pltpu.VMEM((BC * CHUNK,), jnp.int32),
        pltpu.VMEM((BC, CHUNK), jnp.int32),
        [pltpu.VMEM((CHUNK, D), jnp.float32) for _ in range(NBUF)],
        pltpu.VMEM_SHARED((N_PAD, D), jnp.float32),
        [pltpu.SemaphoreType.DMA for _ in range(NBUF)],
    ],
)
def _sc_agg(p_hbm, src_hbm, dst_hbm, zeros2_hbm, acc_out, srcv, dstm, rows,
            acc_sh, gsem):
    c = lax.axis_index("c")
    s = lax.axis_index("s")
    wid = c * NS + s
    pltpu.sync_copy(zeros2_hbm.at[pl.ds(s * RPT, RPT)],
                    acc_sh.at[pl.ds(s * RPT, RPT)])

    pltpu.sync_copy(src_hbm.at[wid, pl.ds(0, BC * CHUNK)], srcv)
    pltpu.sync_copy(dst_hbm.at[wid, pl.ds(0, BC)], dstm)

    def _gather(j, b):
        pltpu.async_copy(
            p_hbm.at[srcv.at[pl.ds(pl.multiple_of(j * CHUNK, 8), CHUNK)]],
            rows[b], gsem[b])

    def _wait(j, b):
        pltpu.make_async_copy(
            p_hbm.at[srcv.at[pl.ds(pl.multiple_of(j * CHUNK, 8), CHUNK)]],
            rows[b], gsem[b]).wait()

    for blk in range(CHUNKS // BC):
        if blk > 0:
            pltpu.sync_copy(
                src_hbm.at[wid, pl.ds(blk * BC * CHUNK, BC * CHUNK)], srcv)
            pltpu.sync_copy(dst_hbm.at[wid, pl.ds(blk * BC, BC)], dstm)
        for b in range(NBUF):
            _gather(b, b)
        if blk == 0:
            plsc.subcore_barrier()

        def group(g, carry):
            for b in range(NBUF):
                j = g * NBUF + b
                _wait(j, b)
                pltpu.sync_copy(rows[b], acc_sh.at[dstm.at[j]], add=True)
                nxt = j + NBUF

                @pl.when(nxt < BC)
                def _():
                    _gather(nxt, b)
            return carry

        lax.fori_loop(0, BC // NBUF, group, 0, unroll=False)
    plsc.subcore_barrier()
    pltpu.sync_copy(acc_sh.at[pl.ds(s * RPT, RPT)],
                    acc_out.at[c, pl.ds(s * RPT, RPT)])


BLK = 1024
GRID = N_PAD // BLK


def _dinv_of(deg_ref):
    deg = deg_ref[0] + deg_ref[1] + 1.0
    return lax.rsqrt(deg)


def _tc_mm_body(x_ref, w_ref, h_ref):
    h_ref[...] = jnp.dot(x_ref[...], w_ref[...],
                         preferred_element_type=jnp.float32)


def _tc_scale_body(deg_ref, h_ref, p_ref):
    p_ref[...] = h_ref[...] * _dinv_of(deg_ref)


def _tc_mid_body(deg_ref, acc_ref, p_ref, b_ref, w_ref, p2_ref):
    dinv = _dinv_of(deg_ref)
    pre = (acc_ref[0] + acc_ref[1] + p_ref[...]) * dinv + b_ref[...]
    mid = jnp.where(pre >= 0.0, pre, 0.01 * pre)
    h = jnp.dot(mid, w_ref[...], preferred_element_type=jnp.float32)
    p2_ref[...] = h * dinv


def _tc_out_body(deg_ref, acc_ref, p_ref, b_ref, o_ref):
    dinv = _dinv_of(deg_ref)
    o_ref[...] = (acc_ref[0] + acc_ref[1] + p_ref[...]) * dinv + b_ref[...]


_deg_spec = pl.BlockSpec((NC, BLK, 1), lambda i: (0, i, 0))
_acc_spec = pl.BlockSpec((NC, BLK, D), lambda i: (0, i, 0))
_row_spec = pl.BlockSpec((BLK, D), lambda i: (i, 0))
_w_spec = pl.BlockSpec((D, D), lambda i: (0, 0))
_b_spec = pl.BlockSpec((1, D), lambda i: (0, 0))
_pD = jax.ShapeDtypeStruct((N_PAD, D), jnp.float32)

_tc_mm = pl.pallas_call(
    _tc_mm_body, grid=(GRID,),
    in_specs=[_row_spec, _w_spec],
    out_specs=_row_spec, out_shape=_pD)

_tc_scale = pl.pallas_call(
    _tc_scale_body, grid=(GRID,),
    in_specs=[_deg_spec, _row_spec],
    out_specs=_row_spec, out_shape=_pD)

_tc_mid = pl.pallas_call(
    _tc_mid_body, grid=(GRID,),
    in_specs=[_deg_spec, _acc_spec, _row_spec, _b_spec, _w_spec],
    out_specs=_row_spec, out_shape=_pD)

_tc_out = pl.pallas_call(
    _tc_out_body, grid=(GRID,),
    in_specs=[_deg_spec, _acc_spec, _row_spec, _b_spec],
    out_specs=_row_spec, out_shape=_pD)


def kernel(x, edge_index, W1, b1, W2, b2):
    src = edge_index[0].astype(jnp.int32)
    dst = edge_index[1].astype(jnp.int32)
    pad = N + jnp.arange(E_PAD - E, dtype=jnp.int32) % (N_PAD - N)
    src2 = jnp.concatenate([src, pad]).reshape(NW, EPW)
    dst3 = jnp.concatenate([dst, pad]).reshape(NW, CHUNKS, CHUNK)
    x_pad = jnp.pad(x.astype(jnp.float32), ((0, N_PAD - N), (0, 0)))
    zeros1 = jnp.zeros((N_PAD,), jnp.float32)
    zeros2 = jnp.zeros((N_PAD, D), jnp.float32)
    ones_c = jnp.ones((CHUNK,), jnp.float32)

    h1 = _tc_mm(x_pad, W1.astype(jnp.float32))
    deg = _sc_degree(dst3, zeros1, ones_c).reshape(NC, N_PAD, 1)
    p1 = _tc_scale(deg, h1)
    acc1 = _sc_agg(p1, src2, dst3, zeros2)
    p2 = _tc_mid(deg, acc1, p1, b1.reshape(1, D).astype(jnp.float32),
                 W2.astype(jnp.float32))
    acc2 = _sc_agg(p2, src2, dst3, zeros2)
    out = _tc_out(deg, acc2, p2, b2.reshape(1, D).astype(jnp.float32))
    return out[:N]


# speedup vs baseline: 1.1181x; 1.0187x over previous
import functools

import jax
import jax.numpy as jnp
from jax import lax
from jax.experimental import pallas as pl
from jax.experimental.pallas import tpu as pltpu
from jax.experimental.pallas import tpu_sc as plsc

N = 10000
D = 128
E = 320000
NC = 2
NS = 16
NW = NC * NS
CHUNK = 64
CHUNKS = 160
BC = 80
NBUF = 4
EPW = CHUNKS * CHUNK
E_PAD = NW * EPW
N_PAD = 10240
RPT = N_PAD // NS

_mesh = plsc.VectorSubcoreMesh(
    core_axis_name="c", subcore_axis_name="s", num_cores=NC, num_subcores=NS
)


@functools.partial(
    pl.kernel,
    out_type=jax.ShapeDtypeStruct((NC * N_PAD,), jnp.float32),
    mesh=_mesh,
    scratch_types=[
        pltpu.VMEM((EPW // 128, 128), jnp.int32),
        pltpu.VMEM((128,), jnp.float32),
        pltpu.VMEM((RPT,), jnp.float32),
        pltpu.VMEM_SHARED((N_PAD,), jnp.float32),
    ],
)
def _sc_degree(dst_hbm, zeros1_hbm, ones_hbm, deg_out, dstm, onesv, stage, deg_sh):
    c = lax.axis_index("c")
    s = lax.axis_index("s")
    wid = c * NS + s
    pltpu.sync_copy(zeros1_hbm.at[pl.ds(s * RPT, RPT)], stage)
    pltpu.sync_copy(stage, deg_sh.at[pl.ds(s * RPT, RPT)])
    pltpu.sync_copy(ones_hbm, onesv)
    pltpu.sync_copy(dst_hbm.at[wid], dstm)
    plsc.subcore_barrier()

    def body(i, carry):
        pltpu.sync_copy(onesv, deg_sh.at[dstm.at[i]], add=True)
        return carry

    lax.fori_loop(0, EPW // 128, body, 0, unroll=False)
    plsc.subcore_barrier()
    pltpu.sync_copy(deg_sh.at[pl.ds(s * RPT, RPT)], stage)
    pltpu.sync_copy(stage, deg_out.at[pl.ds(pl.multiple_of(c * N_PAD + s * RPT, 8), RPT)])


@functools.partial(
    pl.kernel,
    out_type=jax.ShapeDtypeStruct((NC, N_PAD, D), jnp.float32),
    mesh=_mesh,
    scratch_types=[
        pltpu.VMEM((BC * CHUNK,), jnp.int32),
        pltpu.VMEM((BC, CHUNK), jnp.int32),
        [pltpu.VMEM((CHUNK, D), jnp.float32) for _ in range(NBUF)],
        pltpu.VMEM_SHARED((N_PAD, D), jnp.float32),
        [pltpu.SemaphoreType.DMA for _ in range(NBUF)],
    ],
)
def _sc_agg(p_hbm, src_hbm, dst_hbm, zeros2_hbm, acc_out, srcv, dstm, rows,
            acc_sh, gsem):
    c = lax.axis_index("c")
    s = lax.axis_index("s")
    wid = c * NS + s
    pltpu.sync_copy(zeros2_hbm.at[pl.ds(s * RPT, RPT)],
                    acc_sh.at[pl.ds(s * RPT, RPT)])

    pltpu.sync_copy(src_hbm.at[wid, pl.ds(0, BC * CHUNK)], srcv)
    pltpu.sync_copy(dst_hbm.at[wid, pl.ds(0, BC)], dstm)

    def _gather(j, b):
        pltpu.async_copy(
            p_hbm.at[srcv.at[pl.ds(pl.multiple_of(j * CHUNK, 8), CHUNK)]],
            rows[b], gsem[b])

    def _wait(j, b):
        pltpu.make_async_copy(
            p_hbm.at[srcv.at[pl.ds(pl.multiple_of(j * CHUNK, 8), CHUNK)]],
            rows[b], gsem[b]).wait()

    for blk in range(CHUNKS // BC):
        if blk > 0:
            pltpu.sync_copy(
                src_hbm.at[wid, pl.ds(blk * BC * CHUNK, BC * CHUNK)], srcv)
            pltpu.sync_copy(dst_hbm.at[wid, pl.ds(blk * BC, BC)], dstm)
        for b in range(NBUF):
            _gather(b, b)
        if blk == 0:
            plsc.subcore_barrier()

        def group(g, carry):
            for b in range(NBUF):
                j = g * NBUF + b
                _wait(j, b)
                pltpu.sync_copy(rows[b], acc_sh.at[dstm.at[j]], add=True)
                nxt = j + NBUF

                @pl.when(nxt < BC)
                def _():
                    _gather(nxt, b)
            return carry

        lax.fori_loop(0, BC // NBUF, group, 0, unroll=False)
    plsc.subcore_barrier()
    pltpu.sync_copy(acc_sh.at[pl.ds(s * RPT, RPT)],
                    acc_out.at[c, pl.ds(s * RPT, RPT)])


BLK = 1024
GRID = N_PAD // BLK


def _dinv_of(deg_ref):
    deg = deg_ref[0] + deg_ref[1] + 1.0
    return lax.rsqrt(deg)


def _tc_scale_mm_body(deg_ref, x_ref, w_ref, p_ref):
    h = jnp.dot(x_ref[...], w_ref[...], preferred_element_type=jnp.float32)
    p_ref[...] = h * _dinv_of(deg_ref)


def _tc_mid_body(deg_ref, acc_ref, p_ref, b_ref, w_ref, p2_ref):
    dinv = _dinv_of(deg_ref)
    pre = (acc_ref[0] + acc_ref[1] + p_ref[...]) * dinv + b_ref[...]
    mid = jnp.where(pre >= 0.0, pre, 0.01 * pre)
    h = jnp.dot(mid, w_ref[...], preferred_element_type=jnp.float32)
    p2_ref[...] = h * dinv


def _tc_out_body(deg_ref, acc_ref, p_ref, b_ref, o_ref):
    dinv = _dinv_of(deg_ref)
    o_ref[...] = (acc_ref[0] + acc_ref[1] + p_ref[...]) * dinv + b_ref[...]


_deg_spec = pl.BlockSpec((NC, BLK, 1), lambda i: (0, i, 0))
_acc_spec = pl.BlockSpec((NC, BLK, D), lambda i: (0, i, 0))
_row_spec = pl.BlockSpec((BLK, D), lambda i: (i, 0))
_w_spec = pl.BlockSpec((D, D), lambda i: (0, 0))
_b_spec = pl.BlockSpec((1, D), lambda i: (0, 0))
_pD = jax.ShapeDtypeStruct((N_PAD, D), jnp.float32)

_tc_scale_mm = pl.pallas_call(
    _tc_scale_mm_body, grid=(GRID,),
    in_specs=[_deg_spec, _row_spec, _w_spec],
    out_specs=_row_spec, out_shape=_pD)

_tc_mid = pl.pallas_call(
    _tc_mid_body, grid=(GRID,),
    in_specs=[_deg_spec, _acc_spec, _row_spec, _b_spec, _w_spec],
    out_specs=_row_spec, out_shape=_pD)

_tc_out = pl.pallas_call(
    _tc_out_body, grid=(GRID,),
    in_specs=[_deg_spec, _acc_spec, _row_spec, _b_spec],
    out_specs=_row_spec, out_shape=_pD)


def kernel(x, edge_index, W1, b1, W2, b2):
    src = edge_index[0].astype(jnp.int32)
    dst = edge_index[1].astype(jnp.int32)
    pad = N + jnp.arange(E_PAD - E, dtype=jnp.int32) % (N_PAD - N)
    src2 = jnp.concatenate([src, pad]).reshape(NW, EPW)
    dstp = jnp.concatenate([dst, pad])
    dst3 = dstp.reshape(NW, CHUNKS, CHUNK)
    dstd = dstp.reshape(NW, EPW // 128, 128)
    x_pad = jnp.pad(x.astype(jnp.float32), ((0, N_PAD - N), (0, 0)))
    zeros1 = jnp.zeros((N_PAD,), jnp.float32)
    zeros2 = jnp.zeros((N_PAD, D), jnp.float32)
    ones_c = jnp.ones((128,), jnp.float32)

    deg = _sc_degree(dstd, zeros1, ones_c).reshape(NC, N_PAD, 1)
    p1 = _tc_scale_mm(deg, x_pad, W1.astype(jnp.float32))
    acc1 = _sc_agg(p1, src2, dst3, zeros2)
    p2 = _tc_mid(deg, acc1, p1, b1.reshape(1, D).astype(jnp.float32),
                 W2.astype(jnp.float32))
    acc2 = _sc_agg(p2, src2, dst3, zeros2)
    out = _tc_out(deg, acc2, p2, b2.reshape(1, D).astype(jnp.float32))
    return out[:N]
